# baseline (speedup 1.0000x reference)
import functools

import jax
import jax.numpy as jnp
from jax.experimental import pallas as pl
from jax.experimental.pallas import tpu as pltpu

_B, _P, _C = 32, 8732, 21
_PBLK = 1024
_NJ = (_P + _PBLK - 1) // _PBLK
_EPS = 1e-7


def _body(lam_ref, c_ref, t_ref, ci_ref, lo_ref, ls_ref, li_ref,
          o_ab, o_lc, o_rc, o_ll, o_rl, o_wi, o_wl, o_wr):
    b = pl.program_id(0)
    j = pl.program_id(1)

    @pl.when((b == 0) & (j == 0))
    def _init():
        for o in (o_ab, o_lc, o_rc, o_ll, o_rl, o_wi, o_wl, o_wr):
            o[...] = jnp.zeros_like(o)

    lam = lam_ref[0, 0]
    c = c_ref[0]
    t = t_ref[0]
    ci = ci_ref[0]

    lane = jax.lax.broadcasted_iota(jnp.int32, (1, _PBLK), 1)
    valid = (j * _PBLK + lane) < _P
    c = jnp.where(valid, c, 0.5)
    t = jnp.where(valid, t, 0.5)
    ci = jnp.where(valid, ci, 0.5)

    cpe = c + _EPS
    tpe = t + _EPS
    ins = ci + _EPS
    mixed = lam * c + (1.0 - lam) * t + _EPS
    lg_m = jnp.log(mixed)
    lg_i = jnp.log(ins)
    lg_c = jnp.log(cpe)
    lg_t = jnp.log(tpe)
    d_im = lg_i - lg_m
    ab = (ins - mixed) * d_im
    lc = cpe * (lg_c - lg_i)
    rc = tpe * (lg_t - lg_i)
    ab_r = jnp.sum(ab, axis=0, keepdims=True)
    lc_r = jnp.sum(lc, axis=0, keepdims=True)
    rc_r = jnp.sum(rc, axis=0, keepdims=True)

    cmax = jnp.max(c, axis=0, keepdims=True)
    tmax = jnp.max(t, axis=0, keepdims=True)
    left = cmax > c[0:1]
    right = tmax > t[0:1]
    wi = (left & right & valid).astype(jnp.float32)
    wl = (left & ~right & valid).astype(jnp.float32)
    wr = (right & ~left & valid).astype(jnp.float32)

    lo = lo_ref[0]
    ls = ls_ref[0]
    li = li_ref[0]
    dl = jnp.where(valid, li - lo, 0.0)
    dr = jnp.where(valid, li - ls, 0.0)
    ll_r = jnp.sum(dl * dl, axis=0, keepdims=True)
    rl_r = jnp.sum(dr * dr, axis=0, keepdims=True)

    o_ab[...] += ab_r * wi
    o_lc[...] += lc_r * wl
    o_rc[...] += rc_r * wr
    o_ll[...] += ll_r * wl
    o_rl[...] += rl_r * wr
    o_wi[...] += wi
    o_wl[...] += wl
    o_wr[...] += wr


@functools.partial(jax.jit, static_argnames=())
def kernel(conf, conf_flip, loc, loc_flip, conf_shuffle, conf_interpolation,
           loc_shuffle, loc_interpolation, lam):
    del conf_flip, loc_flip
    half = _B // 2
    c_t = jnp.swapaxes(conf, 1, 2)
    t_t = jnp.swapaxes(conf_shuffle, 1, 2)
    ci_t = jnp.swapaxes(conf_interpolation, 1, 2)
    lo_t = jnp.swapaxes(loc, 1, 2)
    ls_t = jnp.swapaxes(loc_shuffle, 1, 2)
    li_t = jnp.swapaxes(loc_interpolation, 1, 2)
    lam_s = jnp.reshape(lam.astype(jnp.float32), (1, 1))

    conf_spec = pl.BlockSpec((1, _C, _PBLK), lambda b, j: (b, 0, j))
    swap_spec = pl.BlockSpec((1, _C, _PBLK), lambda b, j: ((b + half) % _B, 0, j))
    loc_spec = pl.BlockSpec((1, 4, _PBLK), lambda b, j: (b, 0, j))
    lswap_spec = pl.BlockSpec((1, 4, _PBLK), lambda b, j: ((b + half) % _B, 0, j))
    acc_spec = pl.BlockSpec((1, _PBLK), lambda b, j: (0, 0))
    acc_ty = jax.ShapeDtypeStruct((1, _PBLK), jnp.float32)

    outs = pl.pallas_call(
        _body,
        grid=(_B, _NJ),
        in_specs=[
            pl.BlockSpec(memory_space=pltpu.SMEM),
            conf_spec, swap_spec, conf_spec,
            loc_spec, lswap_spec, loc_spec,
        ],
        out_specs=[acc_spec] * 8,
        out_shape=[acc_ty] * 8,
        compiler_params=pltpu.CompilerParams(
            dimension_semantics=("arbitrary", "arbitrary"),
        ),
    )(lam_s, c_t, t_t, ci_t, lo_t, ls_t, li_t)

    s_ab, s_lc, s_rc, s_ll, s_rl, n_i, n_l, n_r = [jnp.sum(o) for o in outs]

    def mmean(s, n):
        return jnp.where(n > 0, s / jnp.maximum(n, 1.0), jnp.float32(0.0))

    total = (mmean(s_ab, n_i) * 0.5
             + mmean(s_lc, n_l) + mmean(s_ll, n_l) * 0.25
             + mmean(s_rc, n_r) + mmean(s_rl, n_r) * 0.25)
    return total
